# baseline (speedup 1.0000x reference)
import functools

import jax
import jax.numpy as jnp
from jax import lax
from jax.experimental import pallas as pl
from jax.experimental.pallas import tpu as pltpu
from jax.experimental.pallas import tpu_sc as plsc

_BETA = 0.25
_BT = 256
_WIN = 2048


def _dist_argmin_body(emb_ref, z_ref, en_out, zn_out, idx_out, esq_ref,
                      e1_ref):
    step = pl.program_id(0)

    @pl.when(step == 0)
    def _():
        e = emb_ref[...]
        n = jnp.sqrt(jnp.sum(e * e, axis=1, keepdims=True))
        en = e / jnp.maximum(n, 1e-12)
        en_out[...] = jnp.concatenate([en, jnp.zeros_like(en)], axis=1)
        esq_ref[...] = jnp.sum(en * en, axis=1).reshape(1, -1)
        e1_ref[...] = en.astype(jnp.bfloat16)

    zb = z_ref[...]
    nz = jnp.sqrt(jnp.sum(zb * zb, axis=1, keepdims=True))
    zn = zb / jnp.maximum(nz, 1e-12)
    zn_out[...] = zn
    zsq = jnp.sum(zn * zn, axis=1, keepdims=True)
    dot = lax.dot_general(zn.astype(jnp.bfloat16), e1_ref[...],
                          (((1,), (1,)), ((), ())),
                          preferred_element_type=jnp.float32)
    d = (zsq + esq_ref[...]) - 2.0 * dot
    n_e = d.shape[1]
    acc_v = jnp.full((d.shape[0], 1), jnp.inf, jnp.float32)
    acc_i = jnp.zeros((d.shape[0], 1), jnp.int32)
    for w in range(0, n_e, _WIN):
        blk = d[:, w:w + _WIN]
        wmin = jnp.min(blk, axis=1, keepdims=True)
        ids = lax.broadcasted_iota(jnp.int32, blk.shape, 1) + w
        widx = jnp.min(jnp.where(blk == wmin, ids, jnp.int32(2**30)),
                       axis=1, keepdims=True)
        take = (wmin < acc_v) | ((wmin == acc_v) & (widx < acc_i))
        acc_v = jnp.where(take, wmin.astype(jnp.bfloat16).astype(jnp.float32),
                          acc_v)
        acc_i = jnp.where(take, widx, acc_i)
    idx_out[...] = acc_i[:, 0]


def _dist_argmin(zf, emb):
    tok, d = zf.shape
    n_e = emb.shape[0]
    grid = tok // _BT
    return pl.pallas_call(
        _dist_argmin_body,
        grid=(grid,),
        in_specs=[
            pl.BlockSpec((n_e, d), lambda i: (0, 0)),
            pl.BlockSpec((_BT, d), lambda i: (i, 0)),
        ],
        out_specs=[
            pl.BlockSpec((n_e, 2 * d), lambda i: (0, 0)),
            pl.BlockSpec((_BT, d), lambda i: (i, 0)),
            pl.BlockSpec((_BT,), lambda i: (i,)),
        ],
        out_shape=[
            jax.ShapeDtypeStruct((n_e, 2 * d), jnp.float32),
            jax.ShapeDtypeStruct((tok, d), jnp.float32),
            jax.ShapeDtypeStruct((tok,), jnp.int32),
        ],
        scratch_shapes=[pltpu.VMEM((1, n_e), jnp.float32),
                        pltpu.VMEM((n_e, d), jnp.bfloat16)],
    )(emb, zf)


def _make_sc_gather(n_e, dpad, tok):
    info = plsc.get_sparse_core_info()
    nc, ns = info.num_cores, info.num_subcores
    nw = nc * ns
    rows = tok // nw
    mesh = plsc.VectorSubcoreMesh(core_axis_name="c", subcore_axis_name="s")

    @functools.partial(
        pl.kernel,
        mesh=mesh,
        out_type=jax.ShapeDtypeStruct((tok, dpad), jnp.float32),
        scratch_types=[
            pltpu.VMEM((rows,), jnp.int32),
            pltpu.VMEM((rows, dpad), jnp.float32),
            pltpu.SemaphoreType.DMA,
        ],
    )
    def gather_k(en_hbm, idx_hbm, out_hbm, idx_v, rows_v, sem):
        wid = lax.axis_index("s") * nc + lax.axis_index("c")
        base = wid * rows
        pltpu.sync_copy(idx_hbm.at[pl.ds(base, rows)], idx_v)
        pltpu.async_copy(en_hbm.at[idx_v], rows_v, sem).wait()
        pltpu.sync_copy(rows_v, out_hbm.at[pl.ds(base, rows)])

    return gather_k


def _combine_body(zn_ref, qn_ref, zq_out, loss_out):
    zn = zn_ref[...]
    qn = qn_ref[...][:, : zn.shape[1]]
    diff = qn - zn
    zq_out[...] = zn + diff
    m = jnp.sum(diff * diff) / jnp.float32(zn.size)
    loss_out[...] = jnp.full((1, 1), _BETA * m + m, jnp.float32)


def _combine(zn, qn):
    tok, d = zn.shape
    return pl.pallas_call(
        _combine_body,
        out_shape=[
            jax.ShapeDtypeStruct((tok, d), jnp.float32),
            jax.ShapeDtypeStruct((1, 1), jnp.float32),
        ],
    )(zn, qn)


def kernel(z, embedding_weight):
    b1, b2, d = z.shape
    zf = z.reshape(b1 * b2, d)
    en, zn, idx = _dist_argmin(zf, embedding_weight)
    gather_k = _make_sc_gather(embedding_weight.shape[0], 2 * d, b1 * b2)
    qn = gather_k(en, idx)
    zq, loss = _combine(zn, qn)
    return (zq.reshape(z.shape), loss[0, 0], idx.reshape(b1, b2))


# speedup vs baseline: 1.1095x; 1.1095x over previous
import functools

import jax
import jax.numpy as jnp
from jax import lax
from jax.experimental import pallas as pl
from jax.experimental.pallas import tpu as pltpu
from jax.experimental.pallas import tpu_sc as plsc

_BETA = 0.25
_BT = 1024
_WIN = 2048


def _dist_argmin_body(emb_ref, z_ref, en_out, zn_out, idx_out, esq_ref,
                      e1_ref):
    step = pl.program_id(0)

    @pl.when(step == 0)
    def _():
        e = emb_ref[...]
        n = jnp.sqrt(jnp.sum(e * e, axis=1, keepdims=True))
        en = e / jnp.maximum(n, 1e-12)
        en_out[...] = jnp.concatenate([en, jnp.zeros_like(en)], axis=1)
        esq_ref[...] = jnp.sum(en * en, axis=1).reshape(1, -1)
        e1_ref[...] = en.astype(jnp.bfloat16)

    zb = z_ref[...]
    nz = jnp.sqrt(jnp.sum(zb * zb, axis=1, keepdims=True))
    zn = zb / jnp.maximum(nz, 1e-12)
    zn_out[...] = zn
    zsq = jnp.sum(zn * zn, axis=1, keepdims=True)
    dot = lax.dot_general(zn.astype(jnp.bfloat16), e1_ref[...],
                          (((1,), (1,)), ((), ())),
                          preferred_element_type=jnp.float32)
    d = (zsq + esq_ref[...]) - 2.0 * dot
    n_e = d.shape[1]
    acc_v = jnp.full((d.shape[0], 1), jnp.inf, jnp.float32)
    acc_i = jnp.zeros((d.shape[0], 1), jnp.int32)
    for w in range(0, n_e, _WIN):
        blk = d[:, w:w + _WIN]
        wmin = jnp.min(blk, axis=1, keepdims=True)
        ids = lax.broadcasted_iota(jnp.int32, blk.shape, 1) + w
        widx = jnp.min(jnp.where(blk == wmin, ids, jnp.int32(2**30)),
                       axis=1, keepdims=True)
        take = (wmin < acc_v) | ((wmin == acc_v) & (widx < acc_i))
        acc_v = jnp.where(take, wmin.astype(jnp.bfloat16).astype(jnp.float32),
                          acc_v)
        acc_i = jnp.where(take, widx, acc_i)
    idx_out[...] = acc_i[:, 0]


def _dist_argmin(zf, emb):
    tok, d = zf.shape
    n_e = emb.shape[0]
    grid = tok // _BT
    return pl.pallas_call(
        _dist_argmin_body,
        grid=(grid,),
        in_specs=[
            pl.BlockSpec((n_e, d), lambda i: (0, 0)),
            pl.BlockSpec((_BT, d), lambda i: (i, 0)),
        ],
        out_specs=[
            pl.BlockSpec((n_e, 2 * d), lambda i: (0, 0)),
            pl.BlockSpec((_BT, d), lambda i: (i, 0)),
            pl.BlockSpec((_BT,), lambda i: (i,)),
        ],
        out_shape=[
            jax.ShapeDtypeStruct((n_e, 2 * d), jnp.float32),
            jax.ShapeDtypeStruct((tok, d), jnp.float32),
            jax.ShapeDtypeStruct((tok,), jnp.int32),
        ],
        scratch_shapes=[pltpu.VMEM((1, n_e), jnp.float32),
                        pltpu.VMEM((n_e, d), jnp.bfloat16)],
    )(emb, zf)


def _make_sc_gather(n_e, dpad, tok):
    info = plsc.get_sparse_core_info()
    nc, ns = info.num_cores, info.num_subcores
    nw = nc * ns
    rows = tok // nw
    mesh = plsc.VectorSubcoreMesh(core_axis_name="c", subcore_axis_name="s")

    @functools.partial(
        pl.kernel,
        mesh=mesh,
        out_type=jax.ShapeDtypeStruct((tok, dpad), jnp.float32),
        scratch_types=[
            pltpu.VMEM((rows,), jnp.int32),
            pltpu.VMEM((rows, dpad), jnp.float32),
            pltpu.SemaphoreType.DMA,
        ],
    )
    def gather_k(en_hbm, idx_hbm, out_hbm, idx_v, rows_v, sem):
        wid = lax.axis_index("s") * nc + lax.axis_index("c")
        base = wid * rows
        pltpu.sync_copy(idx_hbm.at[pl.ds(base, rows)], idx_v)
        pltpu.async_copy(en_hbm.at[idx_v], rows_v, sem).wait()
        pltpu.sync_copy(rows_v, out_hbm.at[pl.ds(base, rows)])

    return gather_k


def _combine_body(zn_ref, qn_ref, zq_out, loss_out):
    zn = zn_ref[...]
    qn = qn_ref[...][:, : zn.shape[1]]
    diff = qn - zn
    zq_out[...] = zn + diff
    m = jnp.sum(diff * diff) / jnp.float32(zn.size)
    loss_out[...] = jnp.full((1, 1), _BETA * m + m, jnp.float32)


def _combine(zn, qn):
    tok, d = zn.shape
    return pl.pallas_call(
        _combine_body,
        out_shape=[
            jax.ShapeDtypeStruct((tok, d), jnp.float32),
            jax.ShapeDtypeStruct((1, 1), jnp.float32),
        ],
    )(zn, qn)


def kernel(z, embedding_weight):
    b1, b2, d = z.shape
    zf = z.reshape(b1 * b2, d)
    en, zn, idx = _dist_argmin(zf, embedding_weight)
    gather_k = _make_sc_gather(embedding_weight.shape[0], 2 * d, b1 * b2)
    qn = gather_k(en, idx)
    zq, loss = _combine(zn, qn)
    return (zq.reshape(z.shape), loss[0, 0], idx.reshape(b1, b2))
